# baseline (speedup 1.0000x reference)
import functools

import jax
import jax.numpy as jnp
from jax import lax
from jax.experimental import pallas as pl
from jax.experimental.pallas import tpu as pltpu
from jax.experimental.pallas import tpu_sc as plsc

NUM_CLASS = 100000
DATASET_NUM = 8
CTX_DIM = 512
N_CLS_CTX = 4
N_DM_CTX = 1
B = 1024
SEQ = 77

NC = 2
NS = 16
NW = NC * NS
BPW = B // NW

_mesh = plsc.VectorSubcoreMesh(core_axis_name="c", subcore_axis_name="s")


@functools.partial(
    pl.kernel,
    out_type=jax.ShapeDtypeStruct((B, SEQ, CTX_DIM), jnp.float32),
    mesh=_mesh,
    compiler_params=pltpu.CompilerParams(use_tc_tiling_on_sc=False),
    scratch_types=[
        pltpu.VMEM((BPW,), jnp.int32),
        pltpu.VMEM((BPW,), jnp.int32),
        pltpu.VMEM((BPW, N_CLS_CTX, CTX_DIM), jnp.float32),
        pltpu.VMEM((BPW, N_DM_CTX, CTX_DIM), jnp.float32),
        pltpu.VMEM((5, CTX_DIM), jnp.float32),
        pltpu.VMEM((2, CTX_DIM), jnp.float32),
        pltpu.VMEM((65, CTX_DIM), jnp.float32),
        pltpu.SemaphoreType.DMA,
        pltpu.SemaphoreType.DMA,
    ],
)
def _assemble(label_h, domain_h, cls_h, dm_h, pref_h, inter_h, suf_h, out_h,
              idx_v, didx_v, rows_v, drows_v, pref_v, inter_v, suf_v,
              gsem, wsem):
    wid = lax.axis_index("s") * NC + lax.axis_index("c")
    base = wid * BPW

    pltpu.sync_copy(label_h.at[pl.ds(base, BPW)], idx_v)
    pltpu.sync_copy(domain_h.at[pl.ds(base, BPW)], didx_v)
    gcls = pltpu.async_copy(cls_h.at[idx_v], rows_v, gsem)
    gdom = pltpu.async_copy(dm_h.at[didx_v], drows_v, gsem)
    pltpu.sync_copy(pref_h.at[0], pref_v)
    pltpu.sync_copy(inter_h.at[0], inter_v)
    pltpu.sync_copy(suf_h.at[0], suf_v)
    gcls.wait()
    gdom.wait()

    pending = []
    for i in range(BPW):
        b = base + i
        pending.extend((
            pltpu.async_copy(pref_v, out_h.at[b, pl.ds(0, 5)], wsem),
            pltpu.async_copy(rows_v.at[i], out_h.at[b, pl.ds(5, N_CLS_CTX)], wsem),
            pltpu.async_copy(inter_v, out_h.at[b, pl.ds(9, 2)], wsem),
            pltpu.async_copy(drows_v.at[i], out_h.at[b, pl.ds(11, N_DM_CTX)], wsem),
            pltpu.async_copy(suf_v, out_h.at[b, pl.ds(12, 65)], wsem),
        ))
    for cp in pending:
        cp.wait()


def kernel(label, domain, clsctx, dmctx, token_prefix_domain,
           token_intermediate_domain, token_suffix_domain):
    return _assemble(label.astype(jnp.int32), domain.astype(jnp.int32),
                     clsctx, dmctx, token_prefix_domain,
                     token_intermediate_domain, token_suffix_domain)


# speedup vs baseline: 4.2715x; 4.2715x over previous
import functools

import jax
import jax.numpy as jnp
from jax import lax
from jax.experimental import pallas as pl
from jax.experimental.pallas import tpu as pltpu
from jax.experimental.pallas import tpu_sc as plsc

NUM_CLASS = 100000
DATASET_NUM = 8
CTX_DIM = 512
B = 1024
SEQ = 77
HEAD = 16
TAIL = SEQ - HEAD

NC = 2
NS = 16
NW = NC * NS
BPW = B // NW
HALF = BPW // 2

_mesh = plsc.VectorSubcoreMesh(core_axis_name="c", subcore_axis_name="s")


@functools.partial(
    pl.kernel,
    out_type=jax.ShapeDtypeStruct((B, SEQ, CTX_DIM), jnp.float32),
    mesh=_mesh,
    scratch_types=[
        pltpu.VMEM((NW, 1, BPW), jnp.int32),
        pltpu.VMEM((NW, 1, BPW), jnp.int32),
        pltpu.VMEM((BPW, 4, CTX_DIM), jnp.float32),
        pltpu.VMEM((BPW, 1, CTX_DIM), jnp.float32),
        pltpu.VMEM((HEAD, CTX_DIM), jnp.float32),
        pltpu.VMEM((HEAD, CTX_DIM), jnp.float32),
        pltpu.VMEM((8, CTX_DIM), jnp.float32),
        pltpu.VMEM_SHARED((TAIL, CTX_DIM), jnp.float32),
        pltpu.SemaphoreType.DMA,
        pltpu.SemaphoreType.DMA,
        pltpu.SemaphoreType.DMA,
        pltpu.SemaphoreType.DMA,
        pltpu.SemaphoreType.DMA,
    ],
)
def _assemble(lab_h, dom_h, cls_h, dm_h, tmpl_h, tail_h, end_h, out_h,
              idx_v, didx_v, rows_v, drows_v, head0, head1, end_v, tail_s,
              gsem_a, gsem_b, hsem0, hsem1, tsem):
    cid = lax.axis_index("c")
    sid = lax.axis_index("s")
    wid = cid * NS + sid
    base = wid * BPW

    pltpu.sync_copy(lab_h.at[wid], idx_v.at[wid])
    pltpu.sync_copy(dom_h.at[wid], didx_v.at[wid])

    iv0 = idx_v[wid, 0, pl.ds(0, HALF)]
    iv1 = idx_v[wid, 0, pl.ds(HALF, HALF)]
    dv0 = didx_v[wid, 0, pl.ds(0, HALF)]
    dv1 = didx_v[wid, 0, pl.ds(HALF, HALF)]
    g0 = pltpu.async_copy(cls_h.at[iv0], rows_v.at[pl.ds(0, HALF)], gsem_a)
    gd0 = pltpu.async_copy(dm_h.at[dv0], drows_v.at[pl.ds(0, HALF)], gsem_a)
    g1 = pltpu.async_copy(cls_h.at[iv1], rows_v.at[pl.ds(HALF, HALF)], gsem_b)
    gd1 = pltpu.async_copy(dm_h.at[dv1], drows_v.at[pl.ds(HALF, HALF)], gsem_b)

    @pl.when(sid == 0)
    def _stage_tail():
        pltpu.sync_copy(tail_h.at[0], tail_s)
    plsc.subcore_barrier()

    pltpu.sync_copy(end_h.at[0], end_v)
    tail_cps = []
    for i in range(BPW):
        tail_cps.append(pltpu.async_copy(
            tail_s.at[pl.ds(0, 56)], out_h.at[base + i, pl.ds(HEAD, 56)], tsem))
        tail_cps.append(pltpu.async_copy(
            end_v.at[pl.ds(0, 5)], out_h.at[base + i, pl.ds(72, 5)], tsem))

    pltpu.sync_copy(tmpl_h.at[0], head0)
    pltpu.sync_copy(tmpl_h.at[0], head1)

    def assemble(head_ref, r):
        for j in range(4):
            for ch in range(CTX_DIM // 16):
                head_ref[5 + j, pl.ds(ch * 16, 16)] = (
                    rows_v[r, j, pl.ds(ch * 16, 16)])
        for ch in range(CTX_DIM // 16):
            head_ref[11, pl.ds(ch * 16, 16)] = drows_v[r, 0, pl.ds(ch * 16, 16)]

    def fire_head(head_ref, r, sem):
        return pltpu.async_copy(head_ref, out_h.at[base + r, pl.ds(0, HEAD)], sem)

    def drain_head(head_ref, sem):
        pltpu.make_async_copy(head_ref, out_h.at[base, pl.ds(0, HEAD)], sem).wait()

    for c, (g, gd, gsem) in enumerate(((g0, gd0, gsem_a), (g1, gd1, gsem_b))):
        g.wait()
        gd.wait()
        r0 = c * HALF
        assemble(head0, r0)
        fire_head(head0, r0, hsem0)
        assemble(head1, r0 + 1)
        fire_head(head1, r0 + 1, hsem1)

        def body(k, _):
            r = r0 + 2 * k
            drain_head(head0, hsem0)
            assemble(head0, r)
            fire_head(head0, r, hsem0)
            drain_head(head1, hsem1)
            assemble(head1, r + 1)
            fire_head(head1, r + 1, hsem1)
            return 0

        lax.fori_loop(1, HALF // 2, body, 0)
        drain_head(head0, hsem0)
        drain_head(head1, hsem1)

    for cp in tail_cps:
        cp.wait()


def kernel(label, domain, clsctx, dmctx, token_prefix_domain,
           token_intermediate_domain, token_suffix_domain):
    lab = label.astype(jnp.int32).reshape(NW, 1, BPW)
    dom = domain.astype(jnp.int32).reshape(NW, 1, BPW)
    tmpl = jnp.concatenate(
        [token_prefix_domain,
         jnp.zeros((1, 4, CTX_DIM), jnp.float32),
         token_intermediate_domain,
         jnp.zeros((1, 1, CTX_DIM), jnp.float32),
         token_suffix_domain[:, :4]], axis=1)
    tail = token_suffix_domain[:, 4:]
    end = jnp.concatenate(
        [token_suffix_domain[:, 60:], jnp.zeros((1, 3, CTX_DIM), jnp.float32)],
        axis=1)
    return _assemble(lab, dom, clsctx, dmctx, tmpl, tail, end)


# speedup vs baseline: 8.4925x; 1.9882x over previous
import functools

import jax
import jax.numpy as jnp
from jax import lax
from jax.experimental import pallas as pl
from jax.experimental.pallas import tpu as pltpu
from jax.experimental.pallas import tpu_sc as plsc

NUM_CLASS = 100000
DATASET_NUM = 8
CTX_DIM = 512
B = 1024
SEQ = 77
REP = 32
NSTATIC = 72

NC = 2
NS = 16
NW = NC * NS
BPW = B // NW
HALF = BPW // 2

SC_PLANES = NSTATIC // NC
STAGE_PER_TILE = SC_PLANES // 12

_mesh = plsc.VectorSubcoreMesh(core_axis_name="c", subcore_axis_name="s")


@functools.partial(
    pl.kernel,
    out_type=jax.ShapeDtypeStruct((SEQ, B, CTX_DIM), jnp.float32),
    mesh=_mesh,
    scratch_types=[
        pltpu.VMEM((1, 1, 2 * BPW), jnp.int32),
        pltpu.VMEM((HALF, 4, CTX_DIM), jnp.float32),
        pltpu.VMEM((HALF, 4, CTX_DIM), jnp.float32),
        pltpu.VMEM((HALF, 1, CTX_DIM), jnp.float32),
        pltpu.VMEM((HALF, CTX_DIM), jnp.float32),
        pltpu.VMEM((HALF, CTX_DIM), jnp.float32),
        pltpu.VMEM_SHARED((SC_PLANES, REP, CTX_DIM), jnp.float32),
        pltpu.SemaphoreType.DMA,
        pltpu.SemaphoreType.DMA,
        pltpu.SemaphoreType.DMA,
        pltpu.SemaphoreType.DMA,
        pltpu.SemaphoreType.DMA,
        pltpu.SemaphoreType.DMA,
    ],
)
def _assemble(idx_h, cls_h, dm_h, tmpl_h, out_h,
              idx_v, rows_a, rows_b, drows_v, stage0, stage1, tmpl_s,
              gsem_a, gsem_b, dsem, ssem0, ssem1, tsem):
    cid = lax.axis_index("c")
    sid = lax.axis_index("s")
    wid = cid * NS + sid
    base = wid * BPW

    pltpu.sync_copy(idx_h.at[wid], idx_v.at[0])
    iv0 = idx_v[0, 0, pl.ds(0, HALF)]
    iv1 = idx_v[0, 0, pl.ds(HALF, HALF)]
    dv0 = idx_v[0, 0, pl.ds(2 * HALF, HALF)]
    dv1 = idx_v[0, 0, pl.ds(3 * HALF, HALF)]
    g0 = pltpu.async_copy(cls_h.at[iv0], rows_a, gsem_a)
    g1 = pltpu.async_copy(cls_h.at[iv1], rows_b, gsem_b)
    gd0 = pltpu.async_copy(dm_h.at[dv0], drows_v, dsem)

    @pl.when(sid < 12)
    def _stage_tmpl():
        src = tmpl_h.at[pl.ds(cid * SC_PLANES + sid * STAGE_PER_TILE,
                              STAGE_PER_TILE)]
        pltpu.sync_copy(src, tmpl_s.at[pl.ds(sid * STAGE_PER_TILE,
                                             STAGE_PER_TILE)])
    plsc.subcore_barrier()

    static_cps = []
    for s in range(SC_PLANES):
        p = cid * SC_PLANES + s
        t = p + jnp.where(p < 5, 0, jnp.where(p < 7, 4, 5))
        for h in range(2):
            static_cps.append(pltpu.async_copy(
                tmpl_s.at[s],
                out_h.at[t, pl.ds(sid * 2 * BPW + h * BPW, BPW)],
                tsem))

    def repack(stage, src, j):
        def row(r, _):
            for ch in range(CTX_DIM // 16):
                stage[r, pl.ds(ch * 16, 16)] = src[r, j, pl.ds(ch * 16, 16)]
            return 0
        lax.fori_loop(0, HALF, row, 0)

    def drain_stage(stage, sem):
        pltpu.make_async_copy(
            stage, out_h.at[0, pl.ds(base, HALF)], sem).wait()

    n = 0

    def do_plane(t, src, j, c):
        nonlocal n
        stage, sem = (stage0, ssem0) if n % 2 == 0 else (stage1, ssem1)
        if n >= 2:
            drain_stage(stage, sem)
        repack(stage, src, j)
        pltpu.async_copy(stage, out_h.at[t, pl.ds(base + c * HALF, HALF)], sem)
        n += 1

    g0.wait()
    for j in range(4):
        do_plane(5 + j, rows_a, j, 0)
    gd0.wait()
    do_plane(11, drows_v, 0, 0)
    gd1 = pltpu.async_copy(dm_h.at[dv1], drows_v, dsem)
    g1.wait()
    for j in range(4):
        do_plane(5 + j, rows_b, j, 1)
    gd1.wait()
    do_plane(11, drows_v, 0, 1)
    drain_stage(stage0, ssem0)
    drain_stage(stage1, ssem1)

    for cp in static_cps:
        cp.wait()


def kernel(label, domain, clsctx, dmctx, token_prefix_domain,
           token_intermediate_domain, token_suffix_domain):
    idx = jnp.concatenate(
        [label.astype(jnp.int32).reshape(NW, 1, BPW),
         domain.astype(jnp.int32).reshape(NW, 1, BPW)], axis=2)
    static_rows = jnp.concatenate(
        [token_prefix_domain, token_intermediate_domain,
         token_suffix_domain], axis=1)
    tmpl = jnp.broadcast_to(static_rows.reshape(NSTATIC, 1, CTX_DIM),
                            (NSTATIC, REP, CTX_DIM))
    res = _assemble(idx, clsctx, dmctx, tmpl)
    return jnp.transpose(res, (1, 0, 2))
